# baseline (speedup 1.0000x reference)
import functools

import jax
import jax.numpy as jnp
from jax import lax
from jax.experimental import pallas as pl
from jax.experimental.pallas import tpu as pltpu
from jax.experimental.pallas import tpu_sc as plsc

H = 256
W = 256
N = 8192
F = 2
B = 4
G = N * F

_SC_INFO = plsc.get_sparse_core_info()
NUM_CORES = _SC_INFO.num_cores
NUM_SUBCORES = _SC_INFO.num_subcores
NW = NUM_CORES * NUM_SUBCORES

NT = 6144
NSC = N - NT
CPW = NSC // NW
NC = 1024
N_CHUNKS = NT // NC

WIN = 32

_INV_SQRT2 = 0.7071067811865476


def _erf(x):
    ax = jnp.abs(x)
    t = 1.0 / (1.0 + 0.3275911 * ax)
    poly = t * (0.254829592 + t * (-0.284496736 + t * (1.421413741
               + t * (-1.453152027 + t * 1.061405429))))
    y = 1.0 - poly * jnp.exp(-ax * ax)
    return jnp.sign(x) * y


def _inv4(t):
    A2323 = t[2][2] * t[3][3] - t[2][3] * t[3][2]
    A1323 = t[2][1] * t[3][3] - t[2][3] * t[3][1]
    A1223 = t[2][1] * t[3][2] - t[2][2] * t[3][1]
    A0323 = t[2][0] * t[3][3] - t[2][3] * t[3][0]
    A0223 = t[2][0] * t[3][2] - t[2][2] * t[3][0]
    A0123 = t[2][0] * t[3][1] - t[2][1] * t[3][0]
    A2313 = t[1][2] * t[3][3] - t[1][3] * t[3][2]
    A1313 = t[1][1] * t[3][3] - t[1][3] * t[3][1]
    A1213 = t[1][1] * t[3][2] - t[1][2] * t[3][1]
    A2312 = t[1][2] * t[2][3] - t[1][3] * t[2][2]
    A1312 = t[1][1] * t[2][3] - t[1][3] * t[2][1]
    A1212 = t[1][1] * t[2][2] - t[1][2] * t[2][1]
    A0313 = t[1][0] * t[3][3] - t[1][3] * t[3][0]
    A0213 = t[1][0] * t[3][2] - t[1][2] * t[3][0]
    A0312 = t[1][0] * t[2][3] - t[1][3] * t[2][0]
    A0212 = t[1][0] * t[2][2] - t[1][2] * t[2][0]
    A0113 = t[1][0] * t[3][1] - t[1][1] * t[3][0]
    A0112 = t[1][0] * t[2][1] - t[1][1] * t[2][0]

    det = (t[0][0] * (t[1][1] * A2323 - t[1][2] * A1323 + t[1][3] * A1223)
           - t[0][1] * (t[1][0] * A2323 - t[1][2] * A0323 + t[1][3] * A0223)
           + t[0][2] * (t[1][0] * A1323 - t[1][1] * A0323 + t[1][3] * A0123)
           - t[0][3] * (t[1][0] * A1223 - t[1][1] * A0223 + t[1][2] * A0123))
    invdet = 1.0 / det
    m = [[None] * 4 for _ in range(4)]
    m[0][0] = (t[1][1] * A2323 - t[1][2] * A1323 + t[1][3] * A1223) * invdet
    m[0][1] = -(t[0][1] * A2323 - t[0][2] * A1323 + t[0][3] * A1223) * invdet
    m[0][2] = (t[0][1] * A2313 - t[0][2] * A1313 + t[0][3] * A1213) * invdet
    m[0][3] = -(t[0][1] * A2312 - t[0][2] * A1312 + t[0][3] * A1212) * invdet
    m[1][0] = -(t[1][0] * A2323 - t[1][2] * A0323 + t[1][3] * A0223) * invdet
    m[1][1] = (t[0][0] * A2323 - t[0][2] * A0323 + t[0][3] * A0223) * invdet
    m[1][2] = -(t[0][0] * A2313 - t[0][2] * A0313 + t[0][3] * A0213) * invdet
    m[1][3] = (t[0][0] * A2312 - t[0][2] * A0312 + t[0][3] * A0212) * invdet
    m[2][0] = (t[1][0] * A1323 - t[1][1] * A0323 + t[1][3] * A0123) * invdet
    m[2][1] = -(t[0][0] * A1323 - t[0][1] * A0323 + t[0][3] * A0123) * invdet
    m[2][2] = (t[0][0] * A1313 - t[0][1] * A0313 + t[0][3] * A0113) * invdet
    m[2][3] = -(t[0][0] * A1312 - t[0][1] * A0312 + t[0][3] * A0112) * invdet
    m[3][0] = -(t[1][0] * A1223 - t[1][1] * A0223 + t[1][2] * A0123) * invdet
    m[3][1] = (t[0][0] * A1223 - t[0][1] * A0223 + t[0][2] * A0123) * invdet
    m[3][2] = -(t[0][0] * A1213 - t[0][1] * A0213 + t[0][2] * A0113) * invdet
    m[3][3] = (t[0][0] * A1212 - t[0][1] * A0212 + t[0][2] * A0112) * invdet
    return m


def _project_body(t_ref, crow_ref, cy_ref, cx_ref):
    b = pl.program_id(0)
    t = [[t_ref[b, i, j] for j in range(4)] for i in range(4)]
    m = _inv4(t)
    xr = crow_ref[0:1, :]
    yr = crow_ref[1:2, :]
    zr = crow_ref[2:3, :]
    cpw = m[3][0] * xr + m[3][1] * yr + m[3][2] * zr + m[3][3]
    cy_ref[0] = (m[0][0] * xr + m[0][1] * yr + m[0][2] * zr + m[0][3]) / cpw
    cx_ref[0] = (m[1][0] * xr + m[1][1] * yr + m[1][2] * zr + m[1][3]) / cpw


def _project(transform_matrix, centers_t):
    return pl.pallas_call(
        _project_body,
        grid=(B,),
        in_specs=[
            pl.BlockSpec(memory_space=pltpu.SMEM),
            pl.BlockSpec((3, N), lambda b: (0, 0)),
        ],
        out_specs=[
            pl.BlockSpec((1, 1, N), lambda b: (b, 0, 0)),
            pl.BlockSpec((1, 1, N), lambda b: (b, 0, 0)),
        ],
        out_shape=[
            jax.ShapeDtypeStruct((B, 1, N), jnp.float32),
            jax.ShapeDtypeStruct((B, 1, N), jnp.float32),
        ],
    )(transform_matrix, centers_t)


def _dense_body(t_ref, crow_ref, ccol_ref, s_ref, st_ref, wt_ref, o_ref):
    b = pl.program_id(0)
    k = pl.program_id(1)
    t = [[t_ref[b, i, j] for j in range(4)] for i in range(4)]
    m = _inv4(t)

    xr = crow_ref[0:1, :]
    yr = crow_ref[1:2, :]
    zr = crow_ref[2:3, :]
    cpw_r = m[3][0] * xr + m[3][1] * yr + m[3][2] * zr + m[3][3]
    cy_r = (m[0][0] * xr + m[0][1] * yr + m[0][2] * zr + m[0][3]) / cpw_r

    xc = ccol_ref[:, 0:1]
    yc = ccol_ref[:, 1:2]
    zc = ccol_ref[:, 2:3]
    cpw_c = m[3][0] * xc + m[3][1] * yc + m[3][2] * zc + m[3][3]
    cx_c = (m[1][0] * xc + m[1][1] * yc + m[1][2] * zc + m[1][3]) / cpw_c

    ys = lax.broadcasted_iota(jnp.int32, (H, 1), 0).astype(jnp.float32)
    xs = lax.broadcasted_iota(jnp.int32, (1, W), 1).astype(jnp.float32)

    acc = jnp.zeros((H, W), jnp.float32)
    for f in range(F):
        s_row = st_ref[f:f + 1, :]
        w_row = wt_ref[f:f + 1, :]
        s_col = s_ref[:, f:f + 1]
        kr = _INV_SQRT2 / s_row
        kc = _INV_SQRT2 / s_col
        iyT = 0.5 * (_erf((ys + 1.0 - cy_r) * kr) - _erf((ys - cy_r) * kr))
        iyT = (iyT * w_row).astype(jnp.bfloat16)
        ix = 0.5 * (_erf((xs + 1.0 - cx_c) * kc) - _erf((xs - cx_c) * kc))
        acc = acc + lax.dot_general(
            iyT, ix.astype(jnp.bfloat16), (((1,), (0,)), ((), ())),
            preferred_element_type=jnp.float32)

    @pl.when(k == 0)
    def _():
        o_ref[...] = jnp.zeros_like(o_ref)

    o_ref[...] += acc[None]


def _dense(transform_matrix, crow, ccol, s, st, wt):
    return pl.pallas_call(
        _dense_body,
        grid=(B, N_CHUNKS),
        in_specs=[
            pl.BlockSpec(memory_space=pltpu.SMEM),
            pl.BlockSpec((3, NC), lambda b, k: (0, k)),
            pl.BlockSpec((NC, 3), lambda b, k: (k, 0)),
            pl.BlockSpec((NC, F), lambda b, k: (k, 0)),
            pl.BlockSpec((F, NC), lambda b, k: (0, k)),
            pl.BlockSpec((F, NC), lambda b, k: (0, k)),
        ],
        out_specs=pl.BlockSpec((1, H, W), lambda b, k: (b, 0, 0)),
        out_shape=jax.ShapeDtypeStruct((B, H, W), jnp.float32),
    )(transform_matrix, crow, ccol, s, st, wt)


def _splat_body(cy_hbm, cx_hbm, s_hbm, w_hbm, out_hbm,
                img, cyb, cxb, sb, wb, wyb):
    wid = lax.axis_index("s") * NUM_CORES + lax.axis_index("c")

    pltpu.sync_copy(s_hbm.at[:, pl.ds(NT + wid * CPW, CPW)], sb)
    pltpu.sync_copy(w_hbm.at[:, pl.ds(NT + wid * CPW, CPW)], wb)

    def batch_body(b, _):
        pltpu.sync_copy(cy_hbm.at[b, 0, pl.ds(NT + wid * CPW, CPW)], cyb)
        pltpu.sync_copy(cx_hbm.at[b, 0, pl.ds(NT + wid * CPW, CPW)], cxb)

        zero = jnp.zeros((16,), jnp.float32)

        def zero_body(i, _):
            for c in range(16):
                img[pl.ds(i * 256 + c * 16, 16)] = zero
            return 0

        lax.fori_loop(0, H * W // 256, zero_body, 0)

        lanei = lax.iota(jnp.int32, 16)

        def group_body(kc, _):
            cy = cyb[pl.ds(kc * 16, 16)]
            cx = cxb[pl.ds(kc * 16, 16)]

            cy = jnp.where(cy != cy, jnp.float32(1e9), cy)
            cx = jnp.where(cx != cx, jnp.float32(1e9), cx)

            cyc = jnp.clip(cy, 0.0, 255.0) + 0.5
            cxc = jnp.clip(cx, 0.0, 255.0) + 0.5
            oy = jnp.clip(cyc.astype(jnp.int32) - WIN // 2, 0, H - WIN)
            ox = jnp.clip(cxc.astype(jnp.int32) - WIN // 2, 0, W - WIN)
            oyf = oy.astype(jnp.float32)
            oxf = ox.astype(jnp.float32)
            base = oy * W + ox

            s0 = sb[0, pl.ds(kc * 16, 16)]
            s1 = sb[1, pl.ds(kc * 16, 16)]
            w0 = wb[0, pl.ds(kc * 16, 16)]
            w1 = wb[1, pl.ds(kc * 16, 16)]
            k0 = _INV_SQRT2 / s0
            k1 = _INV_SQRT2 / s1
            wq0 = 0.25 * w0
            wq1 = 0.25 * w1

            for f, (kk, wq) in enumerate(((k0, wq0), (k1, wq1))):
                a = (oyf - cy) * kk
                e_prev = _erf(a)
                for dy in range(WIN):
                    a = a + kk
                    e = _erf(a)
                    wyb[pl.ds(f * WIN * 16 + dy * 16, 16)] = wq * (e - e_prev)
                    e_prev = e

            r = lax.bitwise_and(lanei - ox, jnp.int32(15))
            moff = [r]
            for j in range(1, 16):
                moff.append(lax.bitwise_and(r + j, jnp.int32(15)))

            rf = r.astype(jnp.float32)
            for h in range(2):
                wxr = [[], []]
                for f, kk in enumerate((k0, k1)):
                    ah0 = (oxf + (16.0 * h) - cx) * kk
                    eh0 = _erf(ah0)
                    a = ah0 + rf * kk
                    e_lo = _erf(a)
                    for j in range(16):
                        e_hi = _erf(a + kk)
                        wxr[f].append(e_hi - e_lo)
                        if j < 15:
                            wrapped = moff[j + 1] == 0
                            a = jnp.where(wrapped, ah0, a + kk)
                            e_lo = jnp.where(wrapped, eh0, e_hi)

                wxr0 = wxr[0]
                wxr1 = wxr[1]

                def row_body(dy, _):
                    vy0 = wyb[pl.ds(dy * 16, 16)]
                    vy1 = wyb[pl.ds(WIN * 16 + dy * 16, 16)]
                    rowb = base + (dy * W + 16 * h)
                    for j in range(16):
                        val = vy0 * wxr0[j] + vy1 * wxr1[j]
                        plsc.addupdate_scatter(img, [rowb + moff[j]], val)
                    return 0

                lax.fori_loop(0, WIN, row_body, 0)
            return 0

        lax.fori_loop(0, CPW // 16, group_body, 0)
        pltpu.sync_copy(img, out_hbm.at[b, wid])
        return 0

    lax.fori_loop(0, B, batch_body, 0)


def _splat(cy, cx, s_flat, w_flat):
    mesh = plsc.VectorSubcoreMesh(core_axis_name="c", subcore_axis_name="s")
    fn = functools.partial(
        pl.kernel,
        out_type=jax.ShapeDtypeStruct((B, NW, H * W), jnp.float32),
        mesh=mesh,
        compiler_params=pltpu.CompilerParams(
            needs_layout_passes=False,
            use_tc_tiling_on_sc=False,
        ),
        scratch_types=[
            pltpu.VMEM((H * W,), jnp.float32),
            pltpu.VMEM((CPW,), jnp.float32),
            pltpu.VMEM((CPW,), jnp.float32),
            pltpu.VMEM((F, CPW), jnp.float32),
            pltpu.VMEM((F, CPW), jnp.float32),
            pltpu.VMEM((F * WIN * 16,), jnp.float32),
        ],
    )(_splat_body)
    return fn(cy, cx, s_flat, w_flat)


def _reduce_body(d_ref, p_ref, o_ref):
    acc = d_ref[0]
    for i in range(NW):
        acc = acc + p_ref[0, i]
    o_ref[0] = acc


def _reduce(dense_img, partials):
    return pl.pallas_call(
        _reduce_body,
        grid=(B,),
        in_specs=[
            pl.BlockSpec((1, H, W), lambda b: (b, 0, 0)),
            pl.BlockSpec((1, NW, H, W), lambda b: (b, 0, 0, 0)),
        ],
        out_specs=pl.BlockSpec((1, H, W), lambda b: (b, 0, 0)),
        out_shape=jax.ShapeDtypeStruct((B, H, W), jnp.float32),
    )(dense_img, partials)


@jax.jit
def _run(transform_matrix, centers, scales, weights):
    centers_t = centers.T
    s_t = scales.T
    w_t = weights.T
    cy, cx = _project(transform_matrix, centers_t)
    partials = _splat(cy, cx, s_t, w_t)
    dense_img = _dense(transform_matrix, centers_t[:, :NT], centers[:NT],
                       scales[:NT], s_t[:, :NT], w_t[:, :NT])
    return _reduce(dense_img, partials.reshape(B, NW, H, W))


def kernel(transform_matrix, centers, scales, weights):
    return _run(transform_matrix, centers, scales, weights)


# speedup vs baseline: 1.1383x; 1.1383x over previous
import functools

import jax
import jax.numpy as jnp
from jax import lax
from jax.experimental import pallas as pl
from jax.experimental.pallas import tpu as pltpu
from jax.experimental.pallas import tpu_sc as plsc

H = 256
W = 256
N = 8192
F = 2
B = 4
G = N * F

_SC_INFO = plsc.get_sparse_core_info()
NUM_CORES = _SC_INFO.num_cores
NUM_SUBCORES = _SC_INFO.num_subcores
NW = NUM_CORES * NUM_SUBCORES

NT = 4096
NSC = N - NT
CPW = NSC // NW
NC = 1024
N_CHUNKS = NT // NC

WIN = 32

_INV_SQRT2 = 0.7071067811865476


def _erf(x):
    ax = jnp.abs(x)
    t = 1.0 / (1.0 + 0.3275911 * ax)
    poly = t * (0.254829592 + t * (-0.284496736 + t * (1.421413741
               + t * (-1.453152027 + t * 1.061405429))))
    y = 1.0 - poly * jnp.exp(-ax * ax)
    return jnp.sign(x) * y


def _inv4(t):
    A2323 = t[2][2] * t[3][3] - t[2][3] * t[3][2]
    A1323 = t[2][1] * t[3][3] - t[2][3] * t[3][1]
    A1223 = t[2][1] * t[3][2] - t[2][2] * t[3][1]
    A0323 = t[2][0] * t[3][3] - t[2][3] * t[3][0]
    A0223 = t[2][0] * t[3][2] - t[2][2] * t[3][0]
    A0123 = t[2][0] * t[3][1] - t[2][1] * t[3][0]
    A2313 = t[1][2] * t[3][3] - t[1][3] * t[3][2]
    A1313 = t[1][1] * t[3][3] - t[1][3] * t[3][1]
    A1213 = t[1][1] * t[3][2] - t[1][2] * t[3][1]
    A2312 = t[1][2] * t[2][3] - t[1][3] * t[2][2]
    A1312 = t[1][1] * t[2][3] - t[1][3] * t[2][1]
    A1212 = t[1][1] * t[2][2] - t[1][2] * t[2][1]
    A0313 = t[1][0] * t[3][3] - t[1][3] * t[3][0]
    A0213 = t[1][0] * t[3][2] - t[1][2] * t[3][0]
    A0312 = t[1][0] * t[2][3] - t[1][3] * t[2][0]
    A0212 = t[1][0] * t[2][2] - t[1][2] * t[2][0]
    A0113 = t[1][0] * t[3][1] - t[1][1] * t[3][0]
    A0112 = t[1][0] * t[2][1] - t[1][1] * t[2][0]

    det = (t[0][0] * (t[1][1] * A2323 - t[1][2] * A1323 + t[1][3] * A1223)
           - t[0][1] * (t[1][0] * A2323 - t[1][2] * A0323 + t[1][3] * A0223)
           + t[0][2] * (t[1][0] * A1323 - t[1][1] * A0323 + t[1][3] * A0123)
           - t[0][3] * (t[1][0] * A1223 - t[1][1] * A0223 + t[1][2] * A0123))
    invdet = 1.0 / det
    m = [[None] * 4 for _ in range(4)]
    m[0][0] = (t[1][1] * A2323 - t[1][2] * A1323 + t[1][3] * A1223) * invdet
    m[0][1] = -(t[0][1] * A2323 - t[0][2] * A1323 + t[0][3] * A1223) * invdet
    m[0][2] = (t[0][1] * A2313 - t[0][2] * A1313 + t[0][3] * A1213) * invdet
    m[0][3] = -(t[0][1] * A2312 - t[0][2] * A1312 + t[0][3] * A1212) * invdet
    m[1][0] = -(t[1][0] * A2323 - t[1][2] * A0323 + t[1][3] * A0223) * invdet
    m[1][1] = (t[0][0] * A2323 - t[0][2] * A0323 + t[0][3] * A0223) * invdet
    m[1][2] = -(t[0][0] * A2313 - t[0][2] * A0313 + t[0][3] * A0213) * invdet
    m[1][3] = (t[0][0] * A2312 - t[0][2] * A0312 + t[0][3] * A0212) * invdet
    m[2][0] = (t[1][0] * A1323 - t[1][1] * A0323 + t[1][3] * A0123) * invdet
    m[2][1] = -(t[0][0] * A1323 - t[0][1] * A0323 + t[0][3] * A0123) * invdet
    m[2][2] = (t[0][0] * A1313 - t[0][1] * A0313 + t[0][3] * A0113) * invdet
    m[2][3] = -(t[0][0] * A1312 - t[0][1] * A0312 + t[0][3] * A0112) * invdet
    m[3][0] = -(t[1][0] * A1223 - t[1][1] * A0223 + t[1][2] * A0123) * invdet
    m[3][1] = (t[0][0] * A1223 - t[0][1] * A0223 + t[0][2] * A0123) * invdet
    m[3][2] = -(t[0][0] * A1213 - t[0][1] * A0213 + t[0][2] * A0113) * invdet
    m[3][3] = (t[0][0] * A1212 - t[0][1] * A0212 + t[0][2] * A0112) * invdet
    return m


def _project_body(t_ref, crow_ref, cy_ref, cx_ref):
    b = pl.program_id(0)
    t = [[t_ref[b, i, j] for j in range(4)] for i in range(4)]
    m = _inv4(t)
    xr = crow_ref[0:1, :]
    yr = crow_ref[1:2, :]
    zr = crow_ref[2:3, :]
    cpw = m[3][0] * xr + m[3][1] * yr + m[3][2] * zr + m[3][3]
    cy_ref[0] = (m[0][0] * xr + m[0][1] * yr + m[0][2] * zr + m[0][3]) / cpw
    cx_ref[0] = (m[1][0] * xr + m[1][1] * yr + m[1][2] * zr + m[1][3]) / cpw


def _project(transform_matrix, centers_t):
    return pl.pallas_call(
        _project_body,
        grid=(B,),
        in_specs=[
            pl.BlockSpec(memory_space=pltpu.SMEM),
            pl.BlockSpec((3, N), lambda b: (0, 0)),
        ],
        out_specs=[
            pl.BlockSpec((1, 1, N), lambda b: (b, 0, 0)),
            pl.BlockSpec((1, 1, N), lambda b: (b, 0, 0)),
        ],
        out_shape=[
            jax.ShapeDtypeStruct((B, 1, N), jnp.float32),
            jax.ShapeDtypeStruct((B, 1, N), jnp.float32),
        ],
    )(transform_matrix, centers_t)


def _dense_body(t_ref, crow_ref, ccol_ref, s_ref, st_ref, wt_ref, o_ref):
    b = pl.program_id(0)
    k = pl.program_id(1)
    t = [[t_ref[b, i, j] for j in range(4)] for i in range(4)]
    m = _inv4(t)

    xr = crow_ref[0:1, :]
    yr = crow_ref[1:2, :]
    zr = crow_ref[2:3, :]
    cpw_r = m[3][0] * xr + m[3][1] * yr + m[3][2] * zr + m[3][3]
    cy_r = (m[0][0] * xr + m[0][1] * yr + m[0][2] * zr + m[0][3]) / cpw_r

    xc = ccol_ref[:, 0:1]
    yc = ccol_ref[:, 1:2]
    zc = ccol_ref[:, 2:3]
    cpw_c = m[3][0] * xc + m[3][1] * yc + m[3][2] * zc + m[3][3]
    cx_c = (m[1][0] * xc + m[1][1] * yc + m[1][2] * zc + m[1][3]) / cpw_c

    ys = lax.broadcasted_iota(jnp.int32, (H, 1), 0).astype(jnp.float32)
    xs = lax.broadcasted_iota(jnp.int32, (1, W), 1).astype(jnp.float32)

    acc = jnp.zeros((H, W), jnp.float32)
    for f in range(F):
        s_row = st_ref[f:f + 1, :]
        w_row = wt_ref[f:f + 1, :]
        s_col = s_ref[:, f:f + 1]
        kr = _INV_SQRT2 / s_row
        kc = _INV_SQRT2 / s_col
        iyT = 0.5 * (_erf((ys + 1.0 - cy_r) * kr) - _erf((ys - cy_r) * kr))
        iyT = (iyT * w_row).astype(jnp.bfloat16)
        ix = 0.5 * (_erf((xs + 1.0 - cx_c) * kc) - _erf((xs - cx_c) * kc))
        acc = acc + lax.dot_general(
            iyT, ix.astype(jnp.bfloat16), (((1,), (0,)), ((), ())),
            preferred_element_type=jnp.float32)

    @pl.when(k == 0)
    def _():
        o_ref[...] = jnp.zeros_like(o_ref)

    o_ref[...] += acc[None]


def _dense(transform_matrix, crow, ccol, s, st, wt):
    return pl.pallas_call(
        _dense_body,
        grid=(B, N_CHUNKS),
        in_specs=[
            pl.BlockSpec(memory_space=pltpu.SMEM),
            pl.BlockSpec((3, NC), lambda b, k: (0, k)),
            pl.BlockSpec((NC, 3), lambda b, k: (k, 0)),
            pl.BlockSpec((NC, F), lambda b, k: (k, 0)),
            pl.BlockSpec((F, NC), lambda b, k: (0, k)),
            pl.BlockSpec((F, NC), lambda b, k: (0, k)),
        ],
        out_specs=pl.BlockSpec((1, H, W), lambda b, k: (b, 0, 0)),
        out_shape=jax.ShapeDtypeStruct((B, H, W), jnp.float32),
    )(transform_matrix, crow, ccol, s, st, wt)


def _splat_body(cy_hbm, cx_hbm, s_hbm, w_hbm, out_hbm,
                img, cyb, cxb, sb, wb, wyb):
    wid = lax.axis_index("s") * NUM_CORES + lax.axis_index("c")

    pltpu.sync_copy(s_hbm.at[:, pl.ds(NT + wid * CPW, CPW)], sb)
    pltpu.sync_copy(w_hbm.at[:, pl.ds(NT + wid * CPW, CPW)], wb)

    def batch_body(b, _):
        pltpu.sync_copy(cy_hbm.at[b, 0, pl.ds(NT + wid * CPW, CPW)], cyb)
        pltpu.sync_copy(cx_hbm.at[b, 0, pl.ds(NT + wid * CPW, CPW)], cxb)

        zero = jnp.zeros((16,), jnp.float32)

        def zero_body(i, _):
            for c in range(16):
                img[pl.ds(i * 256 + c * 16, 16)] = zero
            return 0

        lax.fori_loop(0, H * W // 256, zero_body, 0)

        lanei = lax.iota(jnp.int32, 16)

        def group_body(kc, _):
            cy = cyb[pl.ds(kc * 16, 16)]
            cx = cxb[pl.ds(kc * 16, 16)]

            cy = jnp.where(cy != cy, jnp.float32(1e9), cy)
            cx = jnp.where(cx != cx, jnp.float32(1e9), cx)

            cyc = jnp.clip(cy, 0.0, 255.0) + 0.5
            cxc = jnp.clip(cx, 0.0, 255.0) + 0.5
            oy = jnp.clip(cyc.astype(jnp.int32) - WIN // 2, 0, H - WIN)
            ox = jnp.clip(cxc.astype(jnp.int32) - WIN // 2, 0, W - WIN)
            oyf = oy.astype(jnp.float32)
            oxf = ox.astype(jnp.float32)
            base = oy * W + ox

            s0 = sb[0, pl.ds(kc * 16, 16)]
            s1 = sb[1, pl.ds(kc * 16, 16)]
            w0 = wb[0, pl.ds(kc * 16, 16)]
            w1 = wb[1, pl.ds(kc * 16, 16)]
            k0 = _INV_SQRT2 / s0
            k1 = _INV_SQRT2 / s1
            wq0 = 0.25 * w0
            wq1 = 0.25 * w1

            for f, (kk, wq) in enumerate(((k0, wq0), (k1, wq1))):
                a = (oyf - cy) * kk
                e_prev = _erf(a)
                for dy in range(WIN):
                    a = a + kk
                    e = _erf(a)
                    wyb[pl.ds(f * WIN * 16 + dy * 16, 16)] = wq * (e - e_prev)
                    e_prev = e

            r = lax.bitwise_and(lanei - ox, jnp.int32(15))
            moff = [r]
            for j in range(1, 16):
                moff.append(lax.bitwise_and(r + j, jnp.int32(15)))

            rf = r.astype(jnp.float32)
            for h in range(2):
                wxr = [[], []]
                for f, kk in enumerate((k0, k1)):
                    ah0 = (oxf + (16.0 * h) - cx) * kk
                    eh0 = _erf(ah0)
                    a = ah0 + rf * kk
                    e_lo = _erf(a)
                    for j in range(16):
                        e_hi = _erf(a + kk)
                        wxr[f].append(e_hi - e_lo)
                        if j < 15:
                            wrapped = moff[j + 1] == 0
                            a = jnp.where(wrapped, ah0, a + kk)
                            e_lo = jnp.where(wrapped, eh0, e_hi)

                wxr0 = wxr[0]
                wxr1 = wxr[1]

                def row_body(dy, _):
                    vy0 = wyb[pl.ds(dy * 16, 16)]
                    vy1 = wyb[pl.ds(WIN * 16 + dy * 16, 16)]
                    rowb = base + (dy * W + 16 * h)
                    for j in range(16):
                        val = vy0 * wxr0[j] + vy1 * wxr1[j]
                        plsc.addupdate_scatter(img, [rowb + moff[j]], val)
                    return 0

                lax.fori_loop(0, WIN, row_body, 0)
            return 0

        lax.fori_loop(0, CPW // 16, group_body, 0)
        pltpu.sync_copy(img, out_hbm.at[b, wid])
        return 0

    lax.fori_loop(0, B, batch_body, 0)


def _splat(cy, cx, s_flat, w_flat):
    mesh = plsc.VectorSubcoreMesh(core_axis_name="c", subcore_axis_name="s")
    fn = functools.partial(
        pl.kernel,
        out_type=jax.ShapeDtypeStruct((B, NW, H * W), jnp.float32),
        mesh=mesh,
        compiler_params=pltpu.CompilerParams(
            needs_layout_passes=False,
            use_tc_tiling_on_sc=False,
        ),
        scratch_types=[
            pltpu.VMEM((H * W,), jnp.float32),
            pltpu.VMEM((CPW,), jnp.float32),
            pltpu.VMEM((CPW,), jnp.float32),
            pltpu.VMEM((F, CPW), jnp.float32),
            pltpu.VMEM((F, CPW), jnp.float32),
            pltpu.VMEM((F * WIN * 16,), jnp.float32),
        ],
    )(_splat_body)
    return fn(cy, cx, s_flat, w_flat)


def _reduce_body(d_ref, p_ref, o_ref):
    acc = d_ref[0]
    for i in range(NW):
        acc = acc + p_ref[0, i]
    o_ref[0] = acc


def _reduce(dense_img, partials):
    return pl.pallas_call(
        _reduce_body,
        grid=(B,),
        in_specs=[
            pl.BlockSpec((1, H, W), lambda b: (b, 0, 0)),
            pl.BlockSpec((1, NW, H, W), lambda b: (b, 0, 0, 0)),
        ],
        out_specs=pl.BlockSpec((1, H, W), lambda b: (b, 0, 0)),
        out_shape=jax.ShapeDtypeStruct((B, H, W), jnp.float32),
    )(dense_img, partials)


@jax.jit
def _run(transform_matrix, centers, scales, weights):
    centers_t = centers.T
    s_t = scales.T
    w_t = weights.T
    cy, cx = _project(transform_matrix, centers_t)
    partials = _splat(cy, cx, s_t, w_t)
    dense_img = _dense(transform_matrix, centers_t[:, :NT], centers[:NT],
                       scales[:NT], s_t[:, :NT], w_t[:, :NT])
    return _reduce(dense_img, partials.reshape(B, NW, H, W))


def kernel(transform_matrix, centers, scales, weights):
    return _run(transform_matrix, centers, scales, weights)
